# baseline (speedup 1.0000x reference)
import functools

import jax
import jax.numpy as jnp
from jax import lax
from jax.experimental import pallas as pl
from jax.experimental.pallas import tpu as pltpu
from jax.experimental.pallas import tpu_sc as plsc

NC = 2
NS = 16
NW = NC * NS
A = 10000
A_PAD = 10240
D = 128
PAIRS = 320000
C = 80
CHUNKS_PER_W = PAIRS // (NW * C)
ROWS_PER_TILE = A_PAD // NS
LOAD_ROWS = C
N_LOADS = CHUNKS_PER_W
PW = PAIRS // NW

_mesh = plsc.VectorSubcoreMesh(
    core_axis_name="c", subcore_axis_name="s", num_cores=NC, num_subcores=NS
)


@functools.partial(
    pl.kernel,
    out_type=jax.ShapeDtypeStruct((NC, A_PAD, D), jnp.float32),
    mesh=_mesh,
    scratch_types=[
        pltpu.VMEM((CHUNKS_PER_W, C), jnp.int32),
        pltpu.VMEM((3, LOAD_ROWS, D), jnp.float32),
        pltpu.VMEM_SHARED((A_PAD, D), jnp.float32),
        pltpu.SemaphoreType.DMA((3,)),
        pltpu.SemaphoreType.DMA((3,)),
    ],
)
def _scatter_partials(idx_hbm, inter_hbm, zeros_hbm, out_hbm,
                      idx_v, rows_v, acc_sh, lsem, ssem):
    c = lax.axis_index("c")
    s = lax.axis_index("s")
    w = s * NC + c

    def load_desc(i, b):
        return pltpu.make_async_copy(
            inter_hbm.at[pl.ds(w * PW + i * LOAD_ROWS, LOAD_ROWS)],
            rows_v.at[b], lsem.at[b])

    def scat_desc(i, b):
        return pltpu.make_async_copy(
            rows_v.at[b], acc_sh.at[idx_v.at[i]], ssem.at[b])

    load_desc(0, 0).start()
    load_desc(1, 1).start()
    stripe = pl.ds(s * ROWS_PER_TILE, ROWS_PER_TILE)
    pltpu.sync_copy(zeros_hbm.at[stripe], acc_sh.at[stripe])
    pltpu.sync_copy(idx_hbm.at[w], idx_v)
    plsc.subcore_barrier()

    def body(i, carry):
        b = lax.rem(i, 3)
        load_desc(i, b).wait()
        pltpu.async_copy(rows_v.at[b], acc_sh.at[idx_v.at[i]],
                         ssem.at[b], add=True)
        @pl.when(i + 2 < N_LOADS)
        def _():
            nb = lax.rem(i + 2, 3)
            @pl.when(i >= 1)
            def _():
                scat_desc(i - 1, nb).wait()
            load_desc(i + 2, nb).start()
        return carry

    lax.fori_loop(0, N_LOADS, body, 0)

    for t in (3, 2, 1):
        scat_desc(N_LOADS - t, lax.rem(N_LOADS - t, 3)).wait()

    plsc.subcore_barrier()
    pltpu.sync_copy(acc_sh.at[stripe], out_hbm.at[c, stripe])


def _merge_body(p_ref, o_ref):
    o_ref[...] = p_ref[0] + p_ref[1]


_MERGE_ROWS = 10000


def _merge(partials):
    return pl.pallas_call(
        _merge_body,
        grid=(A // _MERGE_ROWS,),
        in_specs=[pl.BlockSpec((NC, _MERGE_ROWS, D), lambda i: (0, i, 0))],
        out_specs=pl.BlockSpec((_MERGE_ROWS, D), lambda i: (i, 0)),
        out_shape=jax.ShapeDtypeStruct((A, D), jnp.float32),
    )(partials)


def kernel(ind_2, prop, inter):
    idx = ind_2[:, 0].astype(jnp.int32).reshape(NW, CHUNKS_PER_W, C)
    zeros = jnp.zeros((A_PAD, D), jnp.float32)
    partials = _scatter_partials(idx, inter, zeros)
    return _merge(partials)


# speedup vs baseline: 1.0037x; 1.0037x over previous
import functools

import jax
import jax.numpy as jnp
from jax import lax
from jax.experimental import pallas as pl
from jax.experimental.pallas import tpu as pltpu
from jax.experimental.pallas import tpu_sc as plsc

NC = 2
NS = 16
NW = NC * NS
A = 10000
A_PAD = 10240
D = 128
PAIRS = 320000
C = 80
CHUNKS_PER_W = PAIRS // (NW * C)
ROWS_PER_TILE = A_PAD // NS
LOAD_ROWS = C
N_LOADS = CHUNKS_PER_W
PW = PAIRS // NW

_mesh = plsc.VectorSubcoreMesh(
    core_axis_name="c", subcore_axis_name="s", num_cores=NC, num_subcores=NS
)


@functools.partial(
    pl.kernel,
    out_type=jax.ShapeDtypeStruct((NC, A_PAD, D), jnp.float32),
    mesh=_mesh,
    scratch_types=[
        pltpu.VMEM((CHUNKS_PER_W, C), jnp.int32),
        pltpu.VMEM((3, LOAD_ROWS, D), jnp.float32),
        pltpu.VMEM_SHARED((A_PAD, D), jnp.float32),
        pltpu.SemaphoreType.DMA((3,)),
        pltpu.SemaphoreType.DMA((3,)),
    ],
)
def _scatter_partials(idx_hbm, inter_hbm, zeros_hbm, out_hbm,
                      idx_v, rows_v, acc_sh, lsem, ssem):
    c = lax.axis_index("c")
    s = lax.axis_index("s")
    w = s * NC + c

    def load_desc(i, b):
        return pltpu.make_async_copy(
            inter_hbm.at[pl.ds(w * PW + i * LOAD_ROWS, LOAD_ROWS)],
            rows_v.at[b], lsem.at[b])

    def scat_desc(i, b):
        return pltpu.make_async_copy(
            rows_v.at[b], acc_sh.at[idx_v.at[i]], ssem.at[b])

    load_desc(0, 0).start()
    load_desc(1, 1).start()
    stripe = pl.ds(s * ROWS_PER_TILE, ROWS_PER_TILE)
    pltpu.sync_copy(zeros_hbm.at[stripe], acc_sh.at[stripe])
    pltpu.sync_copy(idx_hbm.at[w], idx_v)
    plsc.subcore_barrier()

    def body(i, carry):
        b = lax.rem(i, 3)
        load_desc(i, b).wait()
        pltpu.async_copy(rows_v.at[b], acc_sh.at[idx_v.at[i]],
                         ssem.at[b], add=True)
        @pl.when(i + 2 < N_LOADS)
        def _():
            nb = lax.rem(i + 2, 3)
            @pl.when(i >= 1)
            def _():
                scat_desc(i - 1, nb).wait()
            load_desc(i + 2, nb).start()
        return carry

    lax.fori_loop(0, N_LOADS, body, 0)

    for t in (3, 2, 1):
        scat_desc(N_LOADS - t, lax.rem(N_LOADS - t, 3)).wait()

    plsc.subcore_barrier()
    pltpu.sync_copy(acc_sh.at[stripe], out_hbm.at[c, stripe])


def _merge_body(p_ref, o_ref):
    o_ref[...] = p_ref[0] + p_ref[1]


_MERGE_ROWS = 5000


def _merge(partials):
    return pl.pallas_call(
        _merge_body,
        grid=(A // _MERGE_ROWS,),
        in_specs=[pl.BlockSpec((NC, _MERGE_ROWS, D), lambda i: (0, i, 0))],
        out_specs=pl.BlockSpec((_MERGE_ROWS, D), lambda i: (i, 0)),
        out_shape=jax.ShapeDtypeStruct((A, D), jnp.float32),
    )(partials)


def kernel(ind_2, prop, inter):
    sel = jnp.array([1, 0], jnp.int32)
    idx = jnp.sum(ind_2.astype(jnp.int32) * sel, axis=1).reshape(
        NW, CHUNKS_PER_W, C)
    zeros = jnp.zeros((A_PAD, D), jnp.float32)
    partials = _scatter_partials(idx, inter, zeros)
    return _merge(partials)


# speedup vs baseline: 1.0144x; 1.0107x over previous
import functools

import jax
import jax.numpy as jnp
from jax import lax
from jax.experimental import pallas as pl
from jax.experimental.pallas import tpu as pltpu
from jax.experimental.pallas import tpu_sc as plsc

NC = 2
NS = 16
NW = NC * NS
A = 10000
A_PAD = 10240
D = 128
PAIRS = 320000
C = 80
CHUNKS_PER_W = PAIRS // (NW * C)
ROWS_PER_TILE = A_PAD // NS
LOAD_ROWS = C
N_LOADS = CHUNKS_PER_W
PW = PAIRS // NW

_mesh = plsc.VectorSubcoreMesh(
    core_axis_name="c", subcore_axis_name="s", num_cores=NC, num_subcores=NS
)


@functools.partial(
    pl.kernel,
    out_type=jax.ShapeDtypeStruct((NC, A_PAD, D), jnp.float32),
    mesh=_mesh,
    scratch_types=[
        pltpu.VMEM((CHUNKS_PER_W, C), jnp.int32),
        pltpu.VMEM((3, LOAD_ROWS, D), jnp.float32),
        pltpu.VMEM_SHARED((A_PAD, D), jnp.float32),
        pltpu.SemaphoreType.DMA((3,)),
        pltpu.SemaphoreType.DMA((3,)),
        pltpu.SemaphoreType.DMA((2,)),
    ],
)
def _scatter_partials(idx_hbm, inter_hbm, zeros_hbm, out_hbm,
                      idx_v, rows_v, acc_sh, lsem, ssem, psem):
    c = lax.axis_index("c")
    s = lax.axis_index("s")
    w = s * NC + c

    def load_desc(i, b):
        return pltpu.make_async_copy(
            inter_hbm.at[pl.ds(w * PW + i * LOAD_ROWS, LOAD_ROWS)],
            rows_v.at[b], lsem.at[b])

    def scat_desc(i, b):
        return pltpu.make_async_copy(
            rows_v.at[b], acc_sh.at[idx_v.at[i]], ssem.at[b])

    load_desc(0, 0).start()
    load_desc(1, 1).start()
    stripe = pl.ds(s * ROWS_PER_TILE, ROWS_PER_TILE)
    zdesc = pltpu.make_async_copy(zeros_hbm.at[stripe], acc_sh.at[stripe],
                                  psem.at[0])
    idesc = pltpu.make_async_copy(idx_hbm.at[w], idx_v, psem.at[1])
    zdesc.start()
    idesc.start()
    zdesc.wait()
    idesc.wait()
    plsc.subcore_barrier()

    def body(i, carry):
        b = lax.rem(i, 3)
        load_desc(i, b).wait()
        pltpu.async_copy(rows_v.at[b], acc_sh.at[idx_v.at[i]],
                         ssem.at[b], add=True)
        @pl.when(i + 2 < N_LOADS)
        def _():
            nb = lax.rem(i + 2, 3)
            @pl.when(i >= 1)
            def _():
                scat_desc(i - 1, nb).wait()
            load_desc(i + 2, nb).start()
        return carry

    lax.fori_loop(0, N_LOADS, body, 0)

    for t in (3, 2, 1):
        scat_desc(N_LOADS - t, lax.rem(N_LOADS - t, 3)).wait()

    plsc.subcore_barrier()
    pltpu.sync_copy(acc_sh.at[stripe], out_hbm.at[c, stripe])


def _merge_body(p_ref, o_ref):
    o_ref[...] = p_ref[0] + p_ref[1]


_MERGE_ROWS = 5000


def _merge(partials):
    return pl.pallas_call(
        _merge_body,
        grid=(A // _MERGE_ROWS,),
        in_specs=[pl.BlockSpec((NC, _MERGE_ROWS, D), lambda i: (0, i, 0))],
        out_specs=pl.BlockSpec((_MERGE_ROWS, D), lambda i: (i, 0)),
        out_shape=jax.ShapeDtypeStruct((A, D), jnp.float32),
    )(partials)


def kernel(ind_2, prop, inter):
    idx = ind_2[:, 0].astype(jnp.int32).reshape(NW, CHUNKS_PER_W, C)
    zeros = jnp.zeros((A_PAD, D), jnp.float32)
    partials = _scatter_partials(idx, inter, zeros)
    return _merge(partials)
